# Initial kernel scaffold; baseline (speedup 1.0000x reference)
import jax
import jax.numpy as jnp
from jax.experimental import pallas as pl


def kernel(x, edge_index, W1, b1, W2, b2):
    raise NotImplementedError("write your pallas kernel here")


# speedup vs baseline: 12.4064x; 12.4064x over previous
import functools

import jax
import jax.numpy as jnp
from jax import lax
from jax.experimental import pallas as pl
from jax.experimental.pallas import tpu as pltpu
from jax.experimental.pallas import tpu_sc as plsc

N = 10000
D_IN = 128
HID = 128
N_CLS = 64
E = 320000

N_PAD = 10240
E_PAD = 327680
CHUNK = 128
NROWS = E_PAD // CHUNK
N_SC = 2
N_TILE = 16
BLK = 1024

_MESH = dict(core_axis_name="c", subcore_axis_name="s")


def _zero_vmem(ref, nrows, width):
    z = jnp.zeros((16,), jnp.float32)

    def body(i, _):
        for j in range(width // 16):
            ref[i, pl.ds(j * 16, 16)] = z
        return 0

    lax.fori_loop(0, nrows, body, 0, unroll=False)


def _make_deg():
    ch = NROWS // (N_SC * N_TILE)
    stripe = N_PAD // N_TILE
    mesh = plsc.VectorSubcoreMesh(**_MESH)

    @functools.partial(
        pl.kernel,
        out_type=jax.ShapeDtypeStruct((N_SC, N_PAD, 16), jnp.float32),
        mesh=mesh,
        compiler_params=pltpu.CompilerParams(use_tc_tiling_on_sc=False),
        scratch_types=[
            pltpu.VMEM((ch, CHUNK), jnp.int32),
            pltpu.VMEM((CHUNK, 16), jnp.float32),
            pltpu.VMEM((stripe, 16), jnp.float32),
            pltpu.VMEM_SHARED((N_PAD, 16), jnp.float32),
        ],
    )
    def deg_kernel(dst_hbm, out_hbm, didx, buf, evict, acc):
        c = lax.axis_index("c")
        s = lax.axis_index("s")
        w = c * N_TILE + s
        pltpu.sync_copy(dst_hbm.at[pl.ds(w * ch, ch)], didx)
        _zero_vmem(buf, CHUNK, 16)
        for k in range(stripe // CHUNK):
            pltpu.sync_copy(buf, acc.at[pl.ds(s * stripe + k * CHUNK, CHUNK)])
        plsc.subcore_barrier()
        one = jnp.ones((16,), jnp.float32)

        def fill(i, _):
            buf[i, pl.ds(0, 16)] = one
            return 0

        lax.fori_loop(0, CHUNK, fill, 0, unroll=False)

        def body(j, _):
            pltpu.sync_copy(buf, acc.at[didx.at[j]], add=True)
            return 0

        lax.fori_loop(0, ch, body, 0, unroll=False)
        plsc.subcore_barrier()
        pltpu.sync_copy(acc.at[pl.ds(s * stripe, stripe)], evict)
        pltpu.sync_copy(evict, out_hbm.at[c].at[pl.ds(s * stripe, stripe)])

    return deg_kernel


def _make_prop(width):
    ch = NROWS // N_TILE
    stripe = N_PAD // N_TILE
    mesh = plsc.VectorSubcoreMesh(**_MESH)

    @functools.partial(
        pl.kernel,
        out_type=jax.ShapeDtypeStruct((N_SC, N_PAD, width), jnp.float32),
        mesh=mesh,
        compiler_params=pltpu.CompilerParams(use_tc_tiling_on_sc=False),
        scratch_types=[
            pltpu.VMEM((ch, CHUNK), jnp.int32),
            pltpu.VMEM((ch, CHUNK), jnp.int32),
            pltpu.VMEM((CHUNK, width), jnp.float32),
            pltpu.VMEM((stripe, width), jnp.float32),
            pltpu.VMEM_SHARED((N_PAD, width), jnp.float32),
            pltpu.SemaphoreType.DMA,
        ],
    )
    def prop_kernel(u_hbm, src_hbm, dst_hbm, out_hbm,
                    sidx, didx, rows, evict, acc, sem):
        c = lax.axis_index("c")
        s = lax.axis_index("s")
        pltpu.sync_copy(src_hbm.at[pl.ds(s * ch, ch)], sidx)
        pltpu.sync_copy(dst_hbm.at[pl.ds(s * ch, ch)], didx)
        u_c = u_hbm.at[c]
        _zero_vmem(rows, CHUNK, width)
        for k in range(stripe // CHUNK):
            pltpu.sync_copy(rows, acc.at[pl.ds(s * stripe + k * CHUNK, CHUNK)])
        plsc.subcore_barrier()

        def body(j, _):
            pltpu.async_copy(u_c.at[sidx.at[j]], rows, sem).wait()
            pltpu.sync_copy(rows, acc.at[didx.at[j]], add=True)
            return 0

        lax.fori_loop(0, ch, body, 0, unroll=False)
        plsc.subcore_barrier()
        pltpu.sync_copy(acc.at[pl.ds(s * stripe, stripe)], evict)
        pltpu.sync_copy(evict, out_hbm.at[c].at[pl.ds(s * stripe, stripe)])

    return prop_kernel


def _tc_a(x_pad, w1, deg2):
    def body(x_ref, w_ref, deg_ref, u_ref, dinv_ref):
        d = deg_ref[0, :, 0:1] + deg_ref[1, :, 0:1] + 1.0
        dinv = lax.rsqrt(d)
        y = jnp.dot(x_ref[...], w_ref[...], preferred_element_type=jnp.float32)
        u = y * dinv
        u_ref[0] = u[:, :64]
        u_ref[1] = u[:, 64:]
        dinv_ref[...] = jnp.broadcast_to(dinv, (BLK, 128))

    return pl.pallas_call(
        body,
        grid=(N_PAD // BLK,),
        in_specs=[
            pl.BlockSpec((BLK, 128), lambda i: (i, 0)),
            pl.BlockSpec((128, 128), lambda i: (0, 0)),
            pl.BlockSpec((2, BLK, 16), lambda i: (0, i, 0)),
        ],
        out_specs=[
            pl.BlockSpec((2, BLK, 64), lambda i: (0, i, 0)),
            pl.BlockSpec((BLK, 128), lambda i: (i, 0)),
        ],
        out_shape=[
            jax.ShapeDtypeStruct((2, N_PAD, 64), jnp.float32),
            jax.ShapeDtypeStruct((N_PAD, 128), jnp.float32),
        ],
    )(x_pad, w1, deg2)


def _tc_b(acc1, u1, dinv, w2, b1):
    def body(acc_ref, u1_ref, dinv_ref, w_ref, b_ref, u2_ref):
        s = jnp.concatenate(
            [acc_ref[0] + u1_ref[0], acc_ref[1] + u1_ref[1]], axis=1)
        h = jnp.maximum(dinv_ref[...] * s + b_ref[...], 0.0)
        z = jnp.dot(h, w_ref[...], preferred_element_type=jnp.float32)
        u2 = dinv_ref[:, :64] * z
        u2_ref[0] = u2[:, :32]
        u2_ref[1] = u2[:, 32:]

    return pl.pallas_call(
        body,
        grid=(N_PAD // BLK,),
        in_specs=[
            pl.BlockSpec((2, BLK, 64), lambda i: (0, i, 0)),
            pl.BlockSpec((2, BLK, 64), lambda i: (0, i, 0)),
            pl.BlockSpec((BLK, 128), lambda i: (i, 0)),
            pl.BlockSpec((128, 64), lambda i: (0, 0)),
            pl.BlockSpec((1, 128), lambda i: (0, 0)),
        ],
        out_specs=pl.BlockSpec((2, BLK, 32), lambda i: (0, i, 0)),
        out_shape=jax.ShapeDtypeStruct((2, N_PAD, 32), jnp.float32),
    )(acc1, u1, dinv, w2, b1)


def _tc_c(acc2, u2, dinv, b2):
    def body(acc_ref, u2_ref, dinv_ref, b_ref, out_ref):
        s = jnp.concatenate(
            [acc_ref[0] + u2_ref[0], acc_ref[1] + u2_ref[1]], axis=1)
        out_ref[...] = dinv_ref[:, :64] * s + b_ref[...]

    return pl.pallas_call(
        body,
        grid=(N_PAD // BLK,),
        in_specs=[
            pl.BlockSpec((2, BLK, 32), lambda i: (0, i, 0)),
            pl.BlockSpec((2, BLK, 32), lambda i: (0, i, 0)),
            pl.BlockSpec((BLK, 128), lambda i: (i, 0)),
            pl.BlockSpec((1, 64), lambda i: (0, 0)),
        ],
        out_specs=pl.BlockSpec((BLK, 64), lambda i: (i, 0)),
        out_shape=jax.ShapeDtypeStruct((N_PAD, 64), jnp.float32),
    )(acc2, u2, dinv, b2)


def kernel(x, edge_index, W1, b1, W2, b2):
    pad = jnp.full((E_PAD - E,), N, dtype=jnp.int32)
    src2d = jnp.concatenate([edge_index[0], pad]).reshape(NROWS, CHUNK)
    dst2d = jnp.concatenate([edge_index[1], pad]).reshape(NROWS, CHUNK)
    x_pad = jnp.pad(x, ((0, N_PAD - N), (0, 0)))

    deg2 = _make_deg()(dst2d)
    u1, dinv = _tc_a(x_pad, W1, deg2)
    acc1 = _make_prop(64)(u1, src2d, dst2d)
    u2 = _tc_b(acc1, u1, dinv, W2, b1.reshape(1, HID))
    acc2 = _make_prop(32)(u2, src2d, dst2d)
    out = _tc_c(acc2, u2, dinv, b2.reshape(1, N_CLS))
    return out[:N]


# speedup vs baseline: 15.1948x; 1.2248x over previous
import functools

import jax
import jax.numpy as jnp
from jax import lax
from jax.experimental import pallas as pl
from jax.experimental.pallas import tpu as pltpu
from jax.experimental.pallas import tpu_sc as plsc

N = 10000
D_IN = 128
HID = 128
N_CLS = 64
E = 320000

N_PAD = 10240
E_PAD = 327680
CHUNK = 128
NROWS = E_PAD // CHUNK
N_SC = 2
N_TILE = 16
BLK = 1024

_MESH = dict(core_axis_name="c", subcore_axis_name="s")


def _zero_vmem(ref, nrows, width):
    z = jnp.zeros((16,), jnp.float32)

    def body(i, _):
        for j in range(width // 16):
            ref[i, pl.ds(j * 16, 16)] = z
        return 0

    lax.fori_loop(0, nrows, body, 0, unroll=False)


def _make_deg():
    ch = NROWS // (N_SC * N_TILE)
    stripe = N_PAD // N_TILE
    mesh = plsc.VectorSubcoreMesh(**_MESH)

    @functools.partial(
        pl.kernel,
        out_type=jax.ShapeDtypeStruct((N_SC, N_PAD, 16), jnp.float32),
        mesh=mesh,
        compiler_params=pltpu.CompilerParams(use_tc_tiling_on_sc=False),
        scratch_types=[
            pltpu.VMEM((ch, CHUNK), jnp.int32),
            pltpu.VMEM((CHUNK, 16), jnp.float32),
            pltpu.VMEM_SHARED((N_PAD, 16), jnp.float32),
        ],
    )
    def deg_kernel(dst_hbm, out_hbm, didx, buf, acc):
        c = lax.axis_index("c")
        s = lax.axis_index("s")
        w = c * N_TILE + s
        pltpu.sync_copy(dst_hbm.at[pl.ds(w * ch, ch)], didx)
        _zero_vmem(buf, CHUNK, 16)
        for k in range(stripe // CHUNK):
            pltpu.sync_copy(buf, acc.at[pl.ds(s * stripe + k * CHUNK, CHUNK)])
        plsc.subcore_barrier()
        one = jnp.ones((16,), jnp.float32)

        def fill(i, _):
            buf[i, pl.ds(0, 16)] = one
            return 0

        lax.fori_loop(0, CHUNK, fill, 0, unroll=False)

        def body(j, _):
            pltpu.sync_copy(buf, acc.at[didx.at[j]], add=True)
            return 0

        lax.fori_loop(0, ch, body, 0, unroll=False)
        plsc.subcore_barrier()
        for k in range(stripe // CHUNK):
            r0 = s * stripe + k * CHUNK
            pltpu.sync_copy(acc.at[pl.ds(r0, CHUNK)], buf)
            pltpu.sync_copy(buf, out_hbm.at[c].at[pl.ds(r0, CHUNK)])

    return deg_kernel


def _make_prop(width):
    ch = NROWS // N_TILE
    stripe = N_PAD // N_TILE
    mesh = plsc.VectorSubcoreMesh(**_MESH)

    @functools.partial(
        pl.kernel,
        out_type=jax.ShapeDtypeStruct((N_SC, N_PAD, width), jnp.float32),
        mesh=mesh,
        compiler_params=pltpu.CompilerParams(use_tc_tiling_on_sc=False),
        scratch_types=[
            pltpu.VMEM((ch, CHUNK), jnp.int32),
            pltpu.VMEM((ch, CHUNK), jnp.int32),
            pltpu.VMEM((CHUNK, width), jnp.float32),
            pltpu.VMEM((CHUNK, width), jnp.float32),
            pltpu.VMEM_SHARED((N_PAD, width), jnp.float32),
            pltpu.SemaphoreType.DMA,
            pltpu.SemaphoreType.DMA,
        ],
    )
    def prop_kernel(u_hbm, src_hbm, dst_hbm, out_hbm,
                    sidx, didx, rows0, rows1, acc, sem0, sem1):
        c = lax.axis_index("c")
        s = lax.axis_index("s")
        pltpu.sync_copy(src_hbm.at[pl.ds(s * ch, ch)], sidx)
        pltpu.sync_copy(dst_hbm.at[pl.ds(s * ch, ch)], didx)
        u_c = u_hbm.at[c]
        _zero_vmem(rows0, CHUNK, width)
        for k in range(stripe // CHUNK):
            pltpu.sync_copy(rows0, acc.at[pl.ds(s * stripe + k * CHUNK, CHUNK)])
        plsc.subcore_barrier()

        bufs = ((rows0, sem0), (rows1, sem1))
        pltpu.async_copy(u_c.at[sidx.at[0]], rows0, sem0)
        pltpu.async_copy(u_c.at[sidx.at[1]], rows1, sem1)

        def body(j2, _):
            for b, (rows, sem) in enumerate(bufs):
                j = j2 * 2 + b
                pltpu.make_async_copy(u_c.at[sidx.at[j]], rows, sem).wait()
                pltpu.sync_copy(rows, acc.at[didx.at[j]], add=True)

                @pl.when(j + 2 < ch)
                def _():
                    pltpu.async_copy(u_c.at[sidx.at[j + 2]], rows, sem)
            return 0

        lax.fori_loop(0, ch // 2, body, 0, unroll=False)
        plsc.subcore_barrier()
        for k in range(stripe // CHUNK):
            r0 = s * stripe + k * CHUNK
            pltpu.sync_copy(acc.at[pl.ds(r0, CHUNK)], rows0)
            pltpu.sync_copy(rows0, out_hbm.at[c].at[pl.ds(r0, CHUNK)])

    return prop_kernel


def _tc_a(x_pad, w1, deg2):
    def body(x_ref, w_ref, deg_ref, u_ref, dinv_ref):
        d = deg_ref[0, :, 0:1] + deg_ref[1, :, 0:1] + 1.0
        dinv = lax.rsqrt(d)
        y = jnp.dot(x_ref[...], w_ref[...], preferred_element_type=jnp.float32)
        u = y * dinv
        u_ref[0] = u[:, :64]
        u_ref[1] = u[:, 64:]
        dinv_ref[...] = jnp.broadcast_to(dinv, (BLK, 128))

    return pl.pallas_call(
        body,
        grid=(N_PAD // BLK,),
        in_specs=[
            pl.BlockSpec((BLK, 128), lambda i: (i, 0)),
            pl.BlockSpec((128, 128), lambda i: (0, 0)),
            pl.BlockSpec((2, BLK, 16), lambda i: (0, i, 0)),
        ],
        out_specs=[
            pl.BlockSpec((2, BLK, 64), lambda i: (0, i, 0)),
            pl.BlockSpec((BLK, 128), lambda i: (i, 0)),
        ],
        out_shape=[
            jax.ShapeDtypeStruct((2, N_PAD, 64), jnp.float32),
            jax.ShapeDtypeStruct((N_PAD, 128), jnp.float32),
        ],
    )(x_pad, w1, deg2)


def _tc_b(acc1, u1, dinv, w2, b1):
    def body(acc_ref, u1_ref, dinv_ref, w_ref, b_ref, u2_ref):
        s = jnp.concatenate(
            [acc_ref[0] + u1_ref[0], acc_ref[1] + u1_ref[1]], axis=1)
        h = jnp.maximum(dinv_ref[...] * s + b_ref[...], 0.0)
        z = jnp.dot(h, w_ref[...], preferred_element_type=jnp.float32)
        u2 = dinv_ref[:, :64] * z
        u2_ref[0] = u2[:, :32]
        u2_ref[1] = u2[:, 32:]

    return pl.pallas_call(
        body,
        grid=(N_PAD // BLK,),
        in_specs=[
            pl.BlockSpec((2, BLK, 64), lambda i: (0, i, 0)),
            pl.BlockSpec((2, BLK, 64), lambda i: (0, i, 0)),
            pl.BlockSpec((BLK, 128), lambda i: (i, 0)),
            pl.BlockSpec((128, 64), lambda i: (0, 0)),
            pl.BlockSpec((1, 128), lambda i: (0, 0)),
        ],
        out_specs=pl.BlockSpec((2, BLK, 32), lambda i: (0, i, 0)),
        out_shape=jax.ShapeDtypeStruct((2, N_PAD, 32), jnp.float32),
    )(acc1, u1, dinv, w2, b1)


def _tc_c(acc2, u2, dinv, b2):
    def body(acc_ref, u2_ref, dinv_ref, b_ref, out_ref):
        s = jnp.concatenate(
            [acc_ref[0] + u2_ref[0], acc_ref[1] + u2_ref[1]], axis=1)
        out_ref[...] = dinv_ref[:, :64] * s + b_ref[...]

    return pl.pallas_call(
        body,
        grid=(N_PAD // BLK,),
        in_specs=[
            pl.BlockSpec((2, BLK, 32), lambda i: (0, i, 0)),
            pl.BlockSpec((2, BLK, 32), lambda i: (0, i, 0)),
            pl.BlockSpec((BLK, 128), lambda i: (i, 0)),
            pl.BlockSpec((1, 64), lambda i: (0, 0)),
        ],
        out_specs=pl.BlockSpec((BLK, 64), lambda i: (i, 0)),
        out_shape=jax.ShapeDtypeStruct((N_PAD, 64), jnp.float32),
    )(acc2, u2, dinv, b2)


def kernel(x, edge_index, W1, b1, W2, b2):
    pad = jnp.full((E_PAD - E,), N, dtype=jnp.int32)
    src2d = jnp.concatenate([edge_index[0], pad]).reshape(NROWS, CHUNK)
    dst2d = jnp.concatenate([edge_index[1], pad]).reshape(NROWS, CHUNK)
    x_pad = jnp.pad(x, ((0, N_PAD - N), (0, 0)))

    deg2 = _make_deg()(dst2d)
    u1, dinv = _tc_a(x_pad, W1, deg2)
    acc1 = _make_prop(64)(u1, src2d, dst2d)
    u2 = _tc_b(acc1, u1, dinv, W2, b1.reshape(1, HID))
    acc2 = _make_prop(32)(u2, src2d, dst2d)
    out = _tc_c(acc2, u2, dinv, b2.reshape(1, N_CLS))
    return out[:N]


# speedup vs baseline: 15.2512x; 1.0037x over previous
import functools

import jax
import jax.numpy as jnp
from jax import lax
from jax.experimental import pallas as pl
from jax.experimental.pallas import tpu as pltpu
from jax.experimental.pallas import tpu_sc as plsc

N = 10000
D_IN = 128
HID = 128
N_CLS = 64
E = 320000

N_PAD = 10240
E_PAD = 327680
CHUNK = 128
NROWS = E_PAD // CHUNK
N_SC = 2
N_TILE = 16
BLK = 1024

_MESH = dict(core_axis_name="c", subcore_axis_name="s")


def _zero_vmem(ref, nrows, width):
    z = jnp.zeros((16,), jnp.float32)

    def body(i, _):
        for j in range(width // 16):
            ref[i, pl.ds(j * 16, 16)] = z
        return 0

    lax.fori_loop(0, nrows, body, 0, unroll=False)


def _make_deg():
    ch = NROWS // (N_SC * N_TILE)
    stripe = N_PAD // N_TILE
    mesh = plsc.VectorSubcoreMesh(**_MESH)

    @functools.partial(
        pl.kernel,
        out_type=jax.ShapeDtypeStruct((N_SC, N_PAD, 16), jnp.float32),
        mesh=mesh,
        compiler_params=pltpu.CompilerParams(use_tc_tiling_on_sc=False),
        scratch_types=[
            pltpu.VMEM((ch, CHUNK), jnp.int32),
            pltpu.VMEM((CHUNK, 16), jnp.float32),
            pltpu.VMEM_SHARED((N_PAD, 16), jnp.float32),
        ],
    )
    def deg_kernel(dst_hbm, out_hbm, didx, buf, acc):
        c = lax.axis_index("c")
        s = lax.axis_index("s")
        w = c * N_TILE + s
        pltpu.sync_copy(dst_hbm.at[pl.ds(w * ch, ch)], didx)
        _zero_vmem(buf, CHUNK, 16)
        for k in range(stripe // CHUNK):
            pltpu.sync_copy(buf, acc.at[pl.ds(s * stripe + k * CHUNK, CHUNK)])
        plsc.subcore_barrier()
        one = jnp.ones((16,), jnp.float32)

        def fill(i, _):
            buf[i, pl.ds(0, 16)] = one
            return 0

        lax.fori_loop(0, CHUNK, fill, 0, unroll=False)

        def body(j, _):
            pltpu.sync_copy(buf, acc.at[didx.at[j]], add=True)
            return 0

        lax.fori_loop(0, ch, body, 0, unroll=False)
        plsc.subcore_barrier()
        pltpu.sync_copy(acc.at[pl.ds(s * stripe, stripe)],
                        out_hbm.at[c].at[pl.ds(s * stripe, stripe)])

    return deg_kernel


def _make_prop(width):
    ch = NROWS // N_TILE
    stripe = N_PAD // N_TILE
    mesh = plsc.VectorSubcoreMesh(**_MESH)

    @functools.partial(
        pl.kernel,
        out_type=jax.ShapeDtypeStruct((N_SC, N_PAD, width), jnp.float32),
        mesh=mesh,
        compiler_params=pltpu.CompilerParams(use_tc_tiling_on_sc=False),
        scratch_types=[
            pltpu.VMEM((ch, CHUNK), jnp.int32),
            pltpu.VMEM((ch, CHUNK), jnp.int32),
            pltpu.VMEM((CHUNK, width), jnp.float32),
            pltpu.VMEM((CHUNK, width), jnp.float32),
            pltpu.VMEM_SHARED((N_PAD, width), jnp.float32),
            pltpu.SemaphoreType.DMA,
            pltpu.SemaphoreType.DMA,
        ],
    )
    def prop_kernel(u_hbm, src_hbm, dst_hbm, out_hbm,
                    sidx, didx, rows0, rows1, acc, sem0, sem1):
        c = lax.axis_index("c")
        s = lax.axis_index("s")
        pltpu.sync_copy(src_hbm.at[pl.ds(s * ch, ch)], sidx)
        pltpu.sync_copy(dst_hbm.at[pl.ds(s * ch, ch)], didx)
        u_c = u_hbm.at[c]
        _zero_vmem(rows0, CHUNK, width)
        for k in range(stripe // CHUNK):
            pltpu.async_copy(
                rows0, acc.at[pl.ds(s * stripe + k * CHUNK, CHUNK)], sem0)
        for k in range(stripe // CHUNK):
            pltpu.make_async_copy(
                rows0, acc.at[pl.ds(s * stripe + k * CHUNK, CHUNK)], sem0).wait()
        plsc.subcore_barrier()

        bufs = ((rows0, sem0), (rows1, sem1))
        pltpu.async_copy(u_c.at[sidx.at[0]], rows0, sem0)
        pltpu.async_copy(u_c.at[sidx.at[1]], rows1, sem1)

        def body(j2, _):
            for b, (rows, sem) in enumerate(bufs):
                j = j2 * 2 + b
                pltpu.make_async_copy(u_c.at[sidx.at[j]], rows, sem).wait()
                pltpu.sync_copy(rows, acc.at[didx.at[j]], add=True)

                @pl.when(j + 2 < ch)
                def _():
                    pltpu.async_copy(u_c.at[sidx.at[j + 2]], rows, sem)
            return 0

        lax.fori_loop(0, ch // 2, body, 0, unroll=False)
        plsc.subcore_barrier()
        pltpu.sync_copy(acc.at[pl.ds(s * stripe, stripe)],
                        out_hbm.at[c].at[pl.ds(s * stripe, stripe)])

    return prop_kernel


def _tc_a(x_pad, w1, deg2):
    def body(x_ref, w_ref, deg_ref, u_ref, dinv_ref):
        d = deg_ref[0, :, 0:1] + deg_ref[1, :, 0:1] + 1.0
        dinv = lax.rsqrt(d)
        y = jnp.dot(x_ref[...], w_ref[...], preferred_element_type=jnp.float32)
        u = y * dinv
        u_ref[0] = u[:, :64]
        u_ref[1] = u[:, 64:]
        dinv_ref[...] = jnp.broadcast_to(dinv, (BLK, 128))

    return pl.pallas_call(
        body,
        grid=(N_PAD // BLK,),
        in_specs=[
            pl.BlockSpec((BLK, 128), lambda i: (i, 0)),
            pl.BlockSpec((128, 128), lambda i: (0, 0)),
            pl.BlockSpec((2, BLK, 16), lambda i: (0, i, 0)),
        ],
        out_specs=[
            pl.BlockSpec((2, BLK, 64), lambda i: (0, i, 0)),
            pl.BlockSpec((BLK, 128), lambda i: (i, 0)),
        ],
        out_shape=[
            jax.ShapeDtypeStruct((2, N_PAD, 64), jnp.float32),
            jax.ShapeDtypeStruct((N_PAD, 128), jnp.float32),
        ],
    )(x_pad, w1, deg2)


def _tc_b(acc1, u1, dinv, w2, b1):
    def body(acc_ref, u1_ref, dinv_ref, w_ref, b_ref, u2_ref):
        s = jnp.concatenate(
            [acc_ref[0] + u1_ref[0], acc_ref[1] + u1_ref[1]], axis=1)
        h = jnp.maximum(dinv_ref[...] * s + b_ref[...], 0.0)
        z = jnp.dot(h, w_ref[...], preferred_element_type=jnp.float32)
        u2 = dinv_ref[:, :64] * z
        u2_ref[0] = u2[:, :32]
        u2_ref[1] = u2[:, 32:]

    return pl.pallas_call(
        body,
        grid=(N_PAD // BLK,),
        in_specs=[
            pl.BlockSpec((2, BLK, 64), lambda i: (0, i, 0)),
            pl.BlockSpec((2, BLK, 64), lambda i: (0, i, 0)),
            pl.BlockSpec((BLK, 128), lambda i: (i, 0)),
            pl.BlockSpec((128, 64), lambda i: (0, 0)),
            pl.BlockSpec((1, 128), lambda i: (0, 0)),
        ],
        out_specs=pl.BlockSpec((2, BLK, 32), lambda i: (0, i, 0)),
        out_shape=jax.ShapeDtypeStruct((2, N_PAD, 32), jnp.float32),
    )(acc1, u1, dinv, w2, b1)


def _tc_c(acc2, u2, dinv, b2):
    def body(acc_ref, u2_ref, dinv_ref, b_ref, out_ref):
        s = jnp.concatenate(
            [acc_ref[0] + u2_ref[0], acc_ref[1] + u2_ref[1]], axis=1)
        out_ref[...] = dinv_ref[:, :64] * s + b_ref[...]

    return pl.pallas_call(
        body,
        grid=(N_PAD // BLK,),
        in_specs=[
            pl.BlockSpec((2, BLK, 32), lambda i: (0, i, 0)),
            pl.BlockSpec((2, BLK, 32), lambda i: (0, i, 0)),
            pl.BlockSpec((BLK, 128), lambda i: (i, 0)),
            pl.BlockSpec((1, 64), lambda i: (0, 0)),
        ],
        out_specs=pl.BlockSpec((BLK, 64), lambda i: (i, 0)),
        out_shape=jax.ShapeDtypeStruct((N_PAD, 64), jnp.float32),
    )(acc2, u2, dinv, b2)


def kernel(x, edge_index, W1, b1, W2, b2):
    pad = jnp.full((E_PAD - E,), N, dtype=jnp.int32)
    src2d = jnp.concatenate([edge_index[0], pad]).reshape(NROWS, CHUNK)
    dst2d = jnp.concatenate([edge_index[1], pad]).reshape(NROWS, CHUNK)
    x_pad = jnp.pad(x, ((0, N_PAD - N), (0, 0)))

    deg2 = _make_deg()(dst2d)
    u1, dinv = _tc_a(x_pad, W1, deg2)
    acc1 = _make_prop(64)(u1, src2d, dst2d)
    u2 = _tc_b(acc1, u1, dinv, W2, b1.reshape(1, HID))
    acc2 = _make_prop(32)(u2, src2d, dst2d)
    out = _tc_c(acc2, u2, dinv, b2.reshape(1, N_CLS))
    return out[:N]


# speedup vs baseline: 21.7889x; 1.4287x over previous
import functools

import jax
import jax.numpy as jnp
from jax import lax
from jax.experimental import pallas as pl
from jax.experimental.pallas import tpu as pltpu
from jax.experimental.pallas import tpu_sc as plsc

N = 10000
D_IN = 128
HID = 128
N_CLS = 64
E = 320000

N_PAD = 10240
E_PAD = 327680
CHUNK = 128
NROWS = E_PAD // CHUNK
N_SC = 2
N_TILE = 16
BLK = 1024

_MESH = dict(core_axis_name="c", subcore_axis_name="s")


def _zero_vmem(ref, nrows, width, dtype=jnp.float32):
    lanes = 32 if dtype == jnp.bfloat16 else 16
    z = jnp.zeros((lanes,), dtype)

    def body(i, _):
        for j in range(width // lanes):
            ref[i, pl.ds(j * lanes, lanes)] = z
        return 0

    lax.fori_loop(0, nrows, body, 0, unroll=False)


def _make_deg():
    ch = NROWS // (N_SC * N_TILE)
    stripe = N_PAD // N_TILE
    mesh = plsc.VectorSubcoreMesh(**_MESH)

    @functools.partial(
        pl.kernel,
        out_type=jax.ShapeDtypeStruct((N_SC, N_PAD, 16), jnp.float32),
        mesh=mesh,
        compiler_params=pltpu.CompilerParams(use_tc_tiling_on_sc=False),
        scratch_types=[
            pltpu.VMEM((ch, CHUNK), jnp.int32),
            pltpu.VMEM((CHUNK, 16), jnp.float32),
            pltpu.VMEM_SHARED((N_PAD, 16), jnp.float32),
        ],
    )
    def deg_kernel(dst_hbm, out_hbm, didx, buf, acc):
        c = lax.axis_index("c")
        s = lax.axis_index("s")
        w = c * N_TILE + s
        pltpu.sync_copy(dst_hbm.at[pl.ds(w * ch, ch)], didx)
        _zero_vmem(buf, CHUNK, 16)
        for k in range(stripe // CHUNK):
            pltpu.sync_copy(buf, acc.at[pl.ds(s * stripe + k * CHUNK, CHUNK)])
        plsc.subcore_barrier()
        one = jnp.ones((16,), jnp.float32)

        def fill(i, _):
            buf[i, pl.ds(0, 16)] = one
            return 0

        lax.fori_loop(0, CHUNK, fill, 0, unroll=False)

        def body(j, _):
            pltpu.sync_copy(buf, acc.at[didx.at[j]], add=True)
            return 0

        lax.fori_loop(0, ch, body, 0, unroll=False)
        plsc.subcore_barrier()
        pltpu.sync_copy(acc.at[pl.ds(s * stripe, stripe)],
                        out_hbm.at[c].at[pl.ds(s * stripe, stripe)])

    return deg_kernel


def _make_prop(width, dtype):
    ch = NROWS // N_TILE
    stripe = N_PAD // N_TILE
    mesh = plsc.VectorSubcoreMesh(**_MESH)

    @functools.partial(
        pl.kernel,
        out_type=jax.ShapeDtypeStruct((N_SC, N_PAD, width), dtype),
        mesh=mesh,
        compiler_params=pltpu.CompilerParams(use_tc_tiling_on_sc=False),
        scratch_types=[
            pltpu.VMEM((ch, CHUNK), jnp.int32),
            pltpu.VMEM((ch, CHUNK), jnp.int32),
            pltpu.VMEM((CHUNK, width), dtype),
            pltpu.VMEM((CHUNK, width), dtype),
            pltpu.VMEM_SHARED((N_PAD, width), dtype),
            pltpu.SemaphoreType.DMA,
            pltpu.SemaphoreType.DMA,
        ],
    )
    def prop_kernel(u_hbm, src_hbm, dst_hbm, out_hbm,
                    sidx, didx, rows0, rows1, acc, sem0, sem1):
        c = lax.axis_index("c")
        s = lax.axis_index("s")
        pltpu.sync_copy(src_hbm.at[pl.ds(s * ch, ch)], sidx)
        pltpu.sync_copy(dst_hbm.at[pl.ds(s * ch, ch)], didx)
        u_c = u_hbm.at[c]
        _zero_vmem(rows0, CHUNK, width, dtype)
        for k in range(stripe // CHUNK):
            pltpu.async_copy(
                rows0, acc.at[pl.ds(s * stripe + k * CHUNK, CHUNK)], sem0)
        for k in range(stripe // CHUNK):
            pltpu.make_async_copy(
                rows0, acc.at[pl.ds(s * stripe + k * CHUNK, CHUNK)], sem0).wait()
        plsc.subcore_barrier()

        bufs = ((rows0, sem0), (rows1, sem1))
        pltpu.async_copy(u_c.at[sidx.at[0]], rows0, sem0)
        pltpu.async_copy(u_c.at[sidx.at[1]], rows1, sem1)

        def body(j2, _):
            for b, (rows, sem) in enumerate(bufs):
                j = j2 * 2 + b
                pltpu.make_async_copy(u_c.at[sidx.at[j]], rows, sem).wait()
                pltpu.sync_copy(rows, acc.at[didx.at[j]], add=True)

                @pl.when(j + 2 < ch)
                def _():
                    pltpu.async_copy(u_c.at[sidx.at[j + 2]], rows, sem)
            return 0

        lax.fori_loop(0, ch // 2, body, 0, unroll=False)
        plsc.subcore_barrier()
        pltpu.sync_copy(acc.at[pl.ds(s * stripe, stripe)],
                        out_hbm.at[c].at[pl.ds(s * stripe, stripe)])

    return prop_kernel


def _tc_a(x_pad, w1, deg2):
    def body(x_ref, w_ref, deg_ref, u_ref, dinv_ref):
        d = deg_ref[0, :, 0:1] + deg_ref[1, :, 0:1] + 1.0
        dinv = lax.rsqrt(d)
        y = jnp.dot(x_ref[...], w_ref[...], preferred_element_type=jnp.float32)
        u = y * dinv
        u_ref[0] = u[:, :64].astype(jnp.bfloat16)
        u_ref[1] = u[:, 64:].astype(jnp.bfloat16)
        dinv_ref[...] = jnp.broadcast_to(dinv, (BLK, 128))

    return pl.pallas_call(
        body,
        grid=(N_PAD // BLK,),
        in_specs=[
            pl.BlockSpec((BLK, 128), lambda i: (i, 0)),
            pl.BlockSpec((128, 128), lambda i: (0, 0)),
            pl.BlockSpec((2, BLK, 16), lambda i: (0, i, 0)),
        ],
        out_specs=[
            pl.BlockSpec((2, BLK, 64), lambda i: (0, i, 0)),
            pl.BlockSpec((BLK, 128), lambda i: (i, 0)),
        ],
        out_shape=[
            jax.ShapeDtypeStruct((2, N_PAD, 64), jnp.bfloat16),
            jax.ShapeDtypeStruct((N_PAD, 128), jnp.float32),
        ],
    )(x_pad, w1, deg2)


def _tc_b(acc1, u1, dinv, w2, b1):
    def body(acc_ref, u1_ref, dinv_ref, w_ref, b_ref, u2_ref):
        s = jnp.concatenate(
            [acc_ref[0].astype(jnp.float32) + u1_ref[0].astype(jnp.float32),
             acc_ref[1].astype(jnp.float32) + u1_ref[1].astype(jnp.float32)],
            axis=1)
        h = jnp.maximum(dinv_ref[...] * s + b_ref[...], 0.0)
        z = jnp.dot(h, w_ref[...], preferred_element_type=jnp.float32)
        u2 = dinv_ref[:, :64] * z
        u2_ref[0] = u2[:, :32]
        u2_ref[1] = u2[:, 32:]

    return pl.pallas_call(
        body,
        grid=(N_PAD // BLK,),
        in_specs=[
            pl.BlockSpec((2, BLK, 64), lambda i: (0, i, 0)),
            pl.BlockSpec((2, BLK, 64), lambda i: (0, i, 0)),
            pl.BlockSpec((BLK, 128), lambda i: (i, 0)),
            pl.BlockSpec((128, 64), lambda i: (0, 0)),
            pl.BlockSpec((1, 128), lambda i: (0, 0)),
        ],
        out_specs=pl.BlockSpec((2, BLK, 32), lambda i: (0, i, 0)),
        out_shape=jax.ShapeDtypeStruct((2, N_PAD, 32), jnp.float32),
    )(acc1, u1, dinv, w2, b1)


def _tc_c(acc2, u2, dinv, b2):
    def body(acc_ref, u2_ref, dinv_ref, b_ref, out_ref):
        s = jnp.concatenate(
            [acc_ref[0] + u2_ref[0], acc_ref[1] + u2_ref[1]], axis=1)
        out_ref[...] = dinv_ref[:, :64] * s + b_ref[...]

    return pl.pallas_call(
        body,
        grid=(N_PAD // BLK,),
        in_specs=[
            pl.BlockSpec((2, BLK, 32), lambda i: (0, i, 0)),
            pl.BlockSpec((2, BLK, 32), lambda i: (0, i, 0)),
            pl.BlockSpec((BLK, 128), lambda i: (i, 0)),
            pl.BlockSpec((1, 64), lambda i: (0, 0)),
        ],
        out_specs=pl.BlockSpec((BLK, 64), lambda i: (i, 0)),
        out_shape=jax.ShapeDtypeStruct((N_PAD, 64), jnp.float32),
    )(acc2, u2, dinv, b2)


def kernel(x, edge_index, W1, b1, W2, b2):
    pad = jnp.full((E_PAD - E,), N, dtype=jnp.int32)
    src2d = jnp.concatenate([edge_index[0], pad]).reshape(NROWS, CHUNK)
    dst2d = jnp.concatenate([edge_index[1], pad]).reshape(NROWS, CHUNK)
    x_pad = jnp.pad(x, ((0, N_PAD - N), (0, 0)))

    deg2 = _make_deg()(dst2d)
    u1, dinv = _tc_a(x_pad, W1, deg2)
    acc1 = _make_prop(64, jnp.bfloat16)(u1, src2d, dst2d)
    u2 = _tc_b(acc1, u1, dinv, W2, b1.reshape(1, HID))
    acc2 = _make_prop(32, jnp.float32)(u2, src2d, dst2d)
    out = _tc_c(acc2, u2, dinv, b2.reshape(1, N_CLS))
    return out[:N]


# speedup vs baseline: 24.9165x; 1.1435x over previous
import functools

import jax
import jax.numpy as jnp
from jax import lax
from jax.experimental import pallas as pl
from jax.experimental.pallas import tpu as pltpu
from jax.experimental.pallas import tpu_sc as plsc

N = 10000
D_IN = 128
HID = 128
N_CLS = 64
E = 320000

N_PAD = 10240
E_PAD = 327680
CHUNK = 128
NROWS = E_PAD // CHUNK
N_SC = 2
N_TILE = 16
BLK = 1024

_MESH = dict(core_axis_name="c", subcore_axis_name="s")


def _zero_vmem(ref, nrows, width, dtype=jnp.float32):
    lanes = 32 if dtype == jnp.bfloat16 else 16
    z = jnp.zeros((lanes,), dtype)

    def body(i, _):
        for j in range(width // lanes):
            ref[i, pl.ds(j * lanes, lanes)] = z
        return 0

    lax.fori_loop(0, nrows, body, 0, unroll=False)


def _make_deg():
    ch = NROWS // (N_SC * N_TILE)
    stripe = N_PAD // N_TILE
    mesh = plsc.VectorSubcoreMesh(**_MESH)

    @functools.partial(
        pl.kernel,
        out_type=jax.ShapeDtypeStruct((N_SC, N_PAD, 16), jnp.float32),
        mesh=mesh,
        compiler_params=pltpu.CompilerParams(use_tc_tiling_on_sc=False),
        scratch_types=[
            pltpu.VMEM((ch, CHUNK), jnp.int32),
            pltpu.VMEM((CHUNK, 16), jnp.float32),
            pltpu.VMEM_SHARED((N_PAD, 16), jnp.float32),
        ],
    )
    def deg_kernel(dst_hbm, out_hbm, didx, buf, acc):
        c = lax.axis_index("c")
        s = lax.axis_index("s")
        w = c * N_TILE + s
        pltpu.sync_copy(dst_hbm.at[pl.ds(w * ch, ch)], didx)
        _zero_vmem(buf, CHUNK, 16)
        for k in range(stripe // CHUNK):
            pltpu.sync_copy(buf, acc.at[pl.ds(s * stripe + k * CHUNK, CHUNK)])
        plsc.subcore_barrier()
        one = jnp.ones((16,), jnp.float32)

        def fill(i, _):
            buf[i, pl.ds(0, 16)] = one
            return 0

        lax.fori_loop(0, CHUNK, fill, 0, unroll=False)

        def body(j, _):
            pltpu.sync_copy(buf, acc.at[didx.at[j]], add=True)
            return 0

        lax.fori_loop(0, ch, body, 0, unroll=False)
        plsc.subcore_barrier()
        pltpu.sync_copy(acc.at[pl.ds(s * stripe, stripe)],
                        out_hbm.at[c].at[pl.ds(s * stripe, stripe)])

    return deg_kernel


def _make_prop(width, dtype):
    ch = NROWS // N_TILE
    stripe = N_PAD // N_TILE
    mesh = plsc.VectorSubcoreMesh(**_MESH)

    @functools.partial(
        pl.kernel,
        out_type=jax.ShapeDtypeStruct((N_SC, N_PAD, width), dtype),
        mesh=mesh,
        compiler_params=pltpu.CompilerParams(use_tc_tiling_on_sc=False),
        scratch_types=[
            pltpu.VMEM((ch, CHUNK), jnp.int32),
            pltpu.VMEM((ch, CHUNK), jnp.int32),
            pltpu.VMEM((CHUNK, width), dtype),
            pltpu.VMEM((CHUNK, width), dtype),
            pltpu.VMEM_SHARED((N_PAD, width), dtype),
            pltpu.SemaphoreType.DMA,
            pltpu.SemaphoreType.DMA,
        ],
    )
    def prop_kernel(u_hbm, src_hbm, dst_hbm, out_hbm,
                    sidx, didx, rows0, rows1, acc, sem0, sem1):
        c = lax.axis_index("c")
        s = lax.axis_index("s")
        pltpu.sync_copy(src_hbm.at[pl.ds(s * ch, ch)], sidx)
        pltpu.sync_copy(dst_hbm.at[pl.ds(s * ch, ch)], didx)
        u_c = u_hbm.at[c]
        _zero_vmem(rows0, CHUNK, width, dtype)
        for k in range(stripe // CHUNK):
            pltpu.async_copy(
                rows0, acc.at[pl.ds(s * stripe + k * CHUNK, CHUNK)], sem0)
        for k in range(stripe // CHUNK):
            pltpu.make_async_copy(
                rows0, acc.at[pl.ds(s * stripe + k * CHUNK, CHUNK)], sem0).wait()
        plsc.subcore_barrier()

        bufs = ((rows0, sem0), (rows1, sem1))
        pltpu.async_copy(u_c.at[sidx.at[0]], rows0, sem0)
        pltpu.async_copy(u_c.at[sidx.at[1]], rows1, sem1)

        def body(j2, _):
            for b, (rows, sem) in enumerate(bufs):
                j = j2 * 2 + b
                pltpu.make_async_copy(u_c.at[sidx.at[j]], rows, sem).wait()
                pltpu.sync_copy(rows, acc.at[didx.at[j]], add=True)

                @pl.when(j + 2 < ch)
                def _():
                    pltpu.async_copy(u_c.at[sidx.at[j + 2]], rows, sem)
            return 0

        lax.fori_loop(0, ch // 2, body, 0, unroll=False)
        plsc.subcore_barrier()
        pltpu.sync_copy(acc.at[pl.ds(s * stripe, stripe)],
                        out_hbm.at[c].at[pl.ds(s * stripe, stripe)])

    return prop_kernel


def _tc_a(x_pad, w1, deg2):
    def body(x_ref, w_ref, deg_ref, u_ref, dinv_ref):
        d = deg_ref[0, :, 0:1] + deg_ref[1, :, 0:1] + 1.0
        dinv = lax.rsqrt(d)
        y = jnp.dot(x_ref[...], w_ref[...], preferred_element_type=jnp.float32)
        u = y * dinv
        u_ref[0] = u[:, :64].astype(jnp.bfloat16)
        u_ref[1] = u[:, 64:].astype(jnp.bfloat16)
        dinv_ref[...] = jnp.broadcast_to(dinv, (BLK, 128))

    return pl.pallas_call(
        body,
        grid=(N_PAD // BLK,),
        in_specs=[
            pl.BlockSpec((BLK, 128), lambda i: (i, 0)),
            pl.BlockSpec((128, 128), lambda i: (0, 0)),
            pl.BlockSpec((2, BLK, 16), lambda i: (0, i, 0)),
        ],
        out_specs=[
            pl.BlockSpec((2, BLK, 64), lambda i: (0, i, 0)),
            pl.BlockSpec((BLK, 128), lambda i: (i, 0)),
        ],
        out_shape=[
            jax.ShapeDtypeStruct((2, N_PAD, 64), jnp.bfloat16),
            jax.ShapeDtypeStruct((N_PAD, 128), jnp.float32),
        ],
    )(x_pad, w1, deg2)


def _tc_b(acc1, u1, dinv, w2, b1):
    def body(acc_ref, u1_ref, dinv_ref, w_ref, b_ref, u2_ref):
        s = jnp.concatenate(
            [acc_ref[0].astype(jnp.float32) + u1_ref[0].astype(jnp.float32),
             acc_ref[1].astype(jnp.float32) + u1_ref[1].astype(jnp.float32)],
            axis=1)
        h = jnp.maximum(dinv_ref[...] * s + b_ref[...], 0.0)
        z = jnp.dot(h, w_ref[...], preferred_element_type=jnp.float32)
        u2 = dinv_ref[:, :64] * z
        u2_ref[0] = u2[:, :32].astype(jnp.bfloat16)
        u2_ref[1] = u2[:, 32:].astype(jnp.bfloat16)

    return pl.pallas_call(
        body,
        grid=(N_PAD // BLK,),
        in_specs=[
            pl.BlockSpec((2, BLK, 64), lambda i: (0, i, 0)),
            pl.BlockSpec((2, BLK, 64), lambda i: (0, i, 0)),
            pl.BlockSpec((BLK, 128), lambda i: (i, 0)),
            pl.BlockSpec((128, 64), lambda i: (0, 0)),
            pl.BlockSpec((1, 128), lambda i: (0, 0)),
        ],
        out_specs=pl.BlockSpec((2, BLK, 32), lambda i: (0, i, 0)),
        out_shape=jax.ShapeDtypeStruct((2, N_PAD, 32), jnp.bfloat16),
    )(acc1, u1, dinv, w2, b1)


def _tc_c(acc2, u2, dinv, b2):
    def body(acc_ref, u2_ref, dinv_ref, b_ref, out_ref):
        s = jnp.concatenate(
            [acc_ref[0].astype(jnp.float32) + u2_ref[0].astype(jnp.float32),
             acc_ref[1].astype(jnp.float32) + u2_ref[1].astype(jnp.float32)],
            axis=1)
        out_ref[...] = dinv_ref[:, :64] * s + b_ref[...]

    return pl.pallas_call(
        body,
        grid=(N_PAD // BLK,),
        in_specs=[
            pl.BlockSpec((2, BLK, 32), lambda i: (0, i, 0)),
            pl.BlockSpec((2, BLK, 32), lambda i: (0, i, 0)),
            pl.BlockSpec((BLK, 128), lambda i: (i, 0)),
            pl.BlockSpec((1, 64), lambda i: (0, 0)),
        ],
        out_specs=pl.BlockSpec((BLK, 64), lambda i: (i, 0)),
        out_shape=jax.ShapeDtypeStruct((N_PAD, 64), jnp.float32),
    )(acc2, u2, dinv, b2)


def kernel(x, edge_index, W1, b1, W2, b2):
    pad = jnp.full((E_PAD - E,), N, dtype=jnp.int32)
    src2d = jnp.concatenate([edge_index[0], pad]).reshape(NROWS, CHUNK)
    dst2d = jnp.concatenate([edge_index[1], pad]).reshape(NROWS, CHUNK)
    x_pad = jnp.pad(x, ((0, N_PAD - N), (0, 0)))

    deg2 = _make_deg()(dst2d)
    u1, dinv = _tc_a(x_pad, W1, deg2)
    acc1 = _make_prop(64, jnp.bfloat16)(u1, src2d, dst2d)
    u2 = _tc_b(acc1, u1, dinv, W2, b1.reshape(1, HID))
    acc2 = _make_prop(32, jnp.bfloat16)(u2, src2d, dst2d)
    out = _tc_c(acc2, u2, dinv, b2.reshape(1, N_CLS))
    return out[:N]
